# baseline (speedup 1.0000x reference)
import functools

import jax
import jax.numpy as jnp
from jax import lax
from jax.experimental import pallas as pl
from jax.experimental.pallas import tpu as pltpu
from jax.experimental.pallas import tpu_sc as plsc

N = 10000
E = 320000
D = 128

NC = 2
NS = 16
NW = NC * NS
CPW = D // NW
FL = N * CPW
CH = 6400
L = 16


def _linear_body(f_ref, tw_ref, pw_ref, b_ref, t_ref, a_ref):
    f = f_ref[...]
    t = lax.dot_general(f, tw_ref[...], (((1,), (1,)), ((), ())),
                        preferred_element_type=jnp.float32,
                        precision=lax.Precision.HIGHEST)
    p = lax.dot_general(f, pw_ref[...], (((1,), (1,)), ((), ())),
                        preferred_element_type=jnp.float32,
                        precision=lax.Precision.HIGHEST)
    t_ref[...] = t
    a_ref[...] = t + p + b_ref[...]


def _linear(feat, theta_w, phi_w, bias):
    blk = 400
    return pl.pallas_call(
        _linear_body,
        grid=(N // blk,),
        in_specs=[
            pl.BlockSpec((blk, D), lambda i: (i, 0)),
            pl.BlockSpec((D, D), lambda i: (0, 0)),
            pl.BlockSpec((D, D), lambda i: (0, 0)),
            pl.BlockSpec((1, D), lambda i: (0, 0)),
        ],
        out_specs=[
            pl.BlockSpec((blk, D), lambda i: (i, 0)),
            pl.BlockSpec((blk, D), lambda i: (i, 0)),
        ],
        out_shape=[
            jax.ShapeDtypeStruct((N, D), jnp.float32),
            jax.ShapeDtypeStruct((N, D), jnp.float32),
        ],
    )(feat, theta_w, phi_w, bias)


def _segmin_body(t_hbm, src_hbm, dst_hbm, m_hbm, slab, acc, sbuf, dbuf, idbuf):
    wid = lax.axis_index("s") * NC + lax.axis_index("c")
    pltpu.sync_copy(t_hbm.at[wid], slab)

    inf16 = jnp.full((L,), jnp.inf, jnp.float32)

    def init(i, carry):
        acc[pl.ds(i * L, L)] = inf16
        return carry

    lax.fori_loop(0, FL // L, init, 0)

    iota16 = lax.iota(jnp.int32, L)

    def chunk(ci, carry):
        off = ci * CH
        pltpu.sync_copy(src_hbm.at[pl.ds(off, CH)], sbuf)
        pltpu.sync_copy(dst_hbm.at[pl.ds(off, CH)], dbuf)

        def group(g, carry2):
            src16 = sbuf[pl.ds(g * L, L)]
            dst16 = dbuf[pl.ds(g * L, L)]
            s4 = src16 * CPW
            d4 = dst16 * CPW
            si = [s4 + c for c in range(CPW)]
            di = [d4 + c for c in range(CPW)]
            vals = [plsc.load_gather(slab, [i]) for i in si]
            cur1 = [plsc.load_gather(acc, [i]) for i in di]
            for c in range(CPW):
                plsc.store_scatter(acc, [di[c]],
                                   jnp.minimum(cur1[c], vals[c]))
            plsc.store_scatter(idbuf, [dst16], iota16)
            win = plsc.load_gather(idbuf, [dst16])

            @pl.when(jnp.any(win != iota16))
            def _():
                curs = tuple(plsc.load_gather(acc, [i]) for i in di)
                pend = functools.reduce(
                    lax.bitwise_or,
                    [v < cu for v, cu in zip(vals, curs)])

                def body(st):
                    pend_i, cur_i = st
                    for c in range(CPW):
                        plsc.store_scatter(acc, [di[c]],
                                           jnp.minimum(cur_i[c], vals[c]),
                                           mask=pend_i)
                    re = tuple(plsc.load_gather(acc, [i]) for i in di)
                    ok = functools.reduce(
                        lax.bitwise_and,
                        [r <= v for r, v in zip(re, vals)])
                    return (pend_i & (~ok), re)

                lax.while_loop(lambda s: jnp.any(s[0]), body, (pend, curs))

            return carry2

        lax.fori_loop(0, CH // L, group, 0)
        return carry

    lax.fori_loop(0, E // CH, chunk, 0)
    pltpu.sync_copy(acc, m_hbm.at[wid])


_segmin = functools.partial(
    pl.kernel,
    out_type=jax.ShapeDtypeStruct((NW, FL), jnp.float32),
    mesh=plsc.VectorSubcoreMesh(core_axis_name="c", subcore_axis_name="s"),
    compiler_params=pltpu.CompilerParams(needs_layout_passes=False),
    scratch_types=[
        pltpu.VMEM((FL,), jnp.float32),
        pltpu.VMEM((FL,), jnp.float32),
        pltpu.VMEM((CH,), jnp.int32),
        pltpu.VMEM((CH,), jnp.int32),
        pltpu.VMEM((N,), jnp.int32),
    ],
)(_segmin_body)


def _combine_body(a_ref, m_ref, o_ref):
    a = a_ref[...]
    m = m_ref[...]
    o_ref[...] = jnp.where(jnp.isposinf(m), 0.0, a - m)


def _combine(a, m):
    blk = 400
    return pl.pallas_call(
        _combine_body,
        grid=(N // blk,),
        in_specs=[
            pl.BlockSpec((blk, D), lambda i: (i, 0)),
            pl.BlockSpec((blk, D), lambda i: (i, 0)),
        ],
        out_specs=pl.BlockSpec((blk, D), lambda i: (i, 0)),
        out_shape=jax.ShapeDtypeStruct((N, D), jnp.float32),
    )(a, m)


def kernel(feat, edge_index, theta_w, theta_b, phi_w, phi_b):
    src = edge_index[0]
    dst = edge_index[1]
    bias = (theta_b + phi_b).reshape(1, D)
    t, a = _linear(feat, theta_w, phi_w, bias)
    t32 = t.reshape(N, NW, CPW).transpose(1, 0, 2).reshape(NW, FL)
    m32 = _segmin(t32, src, dst)
    m = m32.reshape(NW, N, CPW).transpose(1, 0, 2).reshape(N, D)
    return _combine(a, m)


# speedup vs baseline: 1.4463x; 1.4463x over previous
import functools

import jax
import jax.numpy as jnp
from jax import lax
from jax.experimental import pallas as pl
from jax.experimental.pallas import tpu as pltpu
from jax.experimental.pallas import tpu_sc as plsc

N = 10000
E = 320000
D = 128

NC = 2
NS = 16
NW = NC * NS
CPW = D // NW
FL = N * CPW
CH = 6400
L = 16


def _linear_body(f_ref, tw_ref, pw_ref, b_ref, t_ref, a_ref):
    f = f_ref[...]
    t = lax.dot_general(f, tw_ref[...], (((1,), (1,)), ((), ())),
                        preferred_element_type=jnp.float32,
                        precision=lax.Precision.HIGHEST)
    p = lax.dot_general(f, pw_ref[...], (((1,), (1,)), ((), ())),
                        preferred_element_type=jnp.float32,
                        precision=lax.Precision.HIGHEST)
    t_ref[...] = t
    a_ref[...] = t + p + b_ref[...]


def _linear(feat, theta_w, phi_w, bias):
    blk = 400
    return pl.pallas_call(
        _linear_body,
        grid=(N // blk,),
        in_specs=[
            pl.BlockSpec((blk, D), lambda i: (i, 0)),
            pl.BlockSpec((D, D), lambda i: (0, 0)),
            pl.BlockSpec((D, D), lambda i: (0, 0)),
            pl.BlockSpec((1, D), lambda i: (0, 0)),
        ],
        out_specs=[
            pl.BlockSpec((blk, D), lambda i: (i, 0)),
            pl.BlockSpec((blk, D), lambda i: (i, 0)),
        ],
        out_shape=[
            jax.ShapeDtypeStruct((N, D), jnp.float32),
            jax.ShapeDtypeStruct((N, D), jnp.float32),
        ],
    )(feat, theta_w, phi_w, bias)


def _segmin_body(t_hbm, src_hbm, dst_hbm, m_hbm, slab, acc, sbuf, dbuf, idbuf):
    wid = lax.axis_index("s") * NC + lax.axis_index("c")
    pltpu.sync_copy(t_hbm.at[wid], slab)

    inf16 = jnp.full((L,), jnp.inf, jnp.float32)

    def init(i, carry):
        acc[pl.ds(i * L, L)] = inf16
        return carry

    lax.fori_loop(0, FL // L, init, 0)

    iota16 = lax.iota(jnp.int32, L)
    false16 = jnp.zeros((L,), jnp.bool_)

    def chunk(ci, carry):
        off = ci * CH
        pltpu.sync_copy(src_hbm.at[pl.ds(off, CH)], sbuf)
        pltpu.sync_copy(dst_hbm.at[pl.ds(off, CH)], dbuf)

        def group(g, sticky):
            src16 = sbuf[pl.ds(g * L, L)]
            dst16 = dbuf[pl.ds(g * L, L)]
            s4 = src16 * CPW
            d4 = dst16 * CPW
            si = [s4 + c for c in range(CPW)]
            di = [d4 + c for c in range(CPW)]
            vals = [plsc.load_gather(slab, [i]) for i in si]
            cur1 = [plsc.load_gather(acc, [i]) for i in di]
            for c in range(CPW):
                plsc.store_scatter(acc, [di[c]],
                                   jnp.minimum(cur1[c], vals[c]))
            cur2 = [plsc.load_gather(acc, [i]) for i in di]
            pend = functools.reduce(
                lax.bitwise_or,
                [v < cu for v, cu in zip(vals, cur2)])
            for c in range(CPW):
                plsc.store_scatter(acc, [di[c]],
                                   jnp.minimum(cur2[c], vals[c]), mask=pend)
            plsc.store_scatter(idbuf, [dst16], iota16)
            w1 = plsc.load_gather(idbuf, [dst16])
            m1 = w1 != iota16
            plsc.store_scatter(idbuf, [dst16], iota16, mask=m1)
            w2 = plsc.load_gather(idbuf, [dst16])
            return sticky | (m1 & (w2 != iota16))

        sticky = lax.fori_loop(0, CH // L, group, false16)

        @pl.when(jnp.any(sticky))
        def _():
            def group_exact(g, carry2):
                s4 = sbuf[pl.ds(g * L, L)] * CPW
                d4 = dbuf[pl.ds(g * L, L)] * CPW
                vals = [plsc.load_gather(slab, [s4 + c]) for c in range(CPW)]
                curs = tuple(plsc.load_gather(acc, [d4 + c])
                             for c in range(CPW))
                pend = functools.reduce(
                    lax.bitwise_or,
                    [v < cu for v, cu in zip(vals, curs)])

                def body(st):
                    pend_i, cur_i = st
                    for c in range(CPW):
                        plsc.store_scatter(acc, [d4 + c],
                                           jnp.minimum(cur_i[c], vals[c]),
                                           mask=pend_i)
                    re = tuple(plsc.load_gather(acc, [d4 + c])
                               for c in range(CPW))
                    ok = functools.reduce(
                        lax.bitwise_and,
                        [r <= v for r, v in zip(re, vals)])
                    return (pend_i & (~ok), re)

                lax.while_loop(lambda s: jnp.any(s[0]), body, (pend, curs))
                return carry2

            lax.fori_loop(0, CH // L, group_exact, 0)

        return carry

    lax.fori_loop(0, E // CH, chunk, 0)
    pltpu.sync_copy(acc, m_hbm.at[wid])


_segmin = functools.partial(
    pl.kernel,
    out_type=jax.ShapeDtypeStruct((NW, FL), jnp.float32),
    mesh=plsc.VectorSubcoreMesh(core_axis_name="c", subcore_axis_name="s"),
    compiler_params=pltpu.CompilerParams(needs_layout_passes=False),
    scratch_types=[
        pltpu.VMEM((FL,), jnp.float32),
        pltpu.VMEM((FL,), jnp.float32),
        pltpu.VMEM((CH,), jnp.int32),
        pltpu.VMEM((CH,), jnp.int32),
        pltpu.VMEM((N,), jnp.int32),
    ],
)(_segmin_body)


def _combine_body(a_ref, m_ref, o_ref):
    a = a_ref[...]
    m = m_ref[...]
    o_ref[...] = jnp.where(jnp.isposinf(m), 0.0, a - m)


def _combine(a, m):
    blk = 400
    return pl.pallas_call(
        _combine_body,
        grid=(N // blk,),
        in_specs=[
            pl.BlockSpec((blk, D), lambda i: (i, 0)),
            pl.BlockSpec((blk, D), lambda i: (i, 0)),
        ],
        out_specs=pl.BlockSpec((blk, D), lambda i: (i, 0)),
        out_shape=jax.ShapeDtypeStruct((N, D), jnp.float32),
    )(a, m)


def kernel(feat, edge_index, theta_w, theta_b, phi_w, phi_b):
    src = edge_index[0]
    dst = edge_index[1]
    bias = (theta_b + phi_b).reshape(1, D)
    t, a = _linear(feat, theta_w, phi_w, bias)
    t32 = t.reshape(N, NW, CPW).transpose(1, 0, 2).reshape(NW, FL)
    m32 = _segmin(t32, src, dst)
    m = m32.reshape(NW, N, CPW).transpose(1, 0, 2).reshape(N, D)
    return _combine(a, m)
